# Initial kernel scaffold; baseline (speedup 1.0000x reference)
import jax
import jax.numpy as jnp
from jax.experimental import pallas as pl


def kernel(h_V, edge_index, h_E, W1, b1, W2, b2, W3, b3, Wf1, bf1, Wf2, bf2, g0, bn0, g1, bn1):
    raise NotImplementedError("write your pallas kernel here")


# speedup vs baseline: 1.4001x; 1.4001x over previous
import functools

import jax
import jax.numpy as jnp
from jax import lax
from jax.experimental import pallas as pl
from jax.experimental.pallas import tpu as pltpu
from jax.experimental.pallas import tpu_sc as plsc

N_NODES = 10000
N_EDGES = 160000
D = 256
EPS = 1e-6

NC = 2
NS = 16
NW = NC * NS


def _sc_gather_rows(table, idx, *, chunk=200):
    n_per_w = N_EDGES // NW
    n_chunks = n_per_w // chunk
    mesh = plsc.VectorSubcoreMesh(
        core_axis_name="c", subcore_axis_name="s", num_cores=NC, num_subcores=NS)

    @functools.partial(
        pl.kernel,
        out_type=jax.ShapeDtypeStruct((N_EDGES, D), jnp.float32),
        mesh=mesh,
        scratch_types=[
            pltpu.VMEM((chunk,), jnp.int32),
            pltpu.VMEM((chunk, D), jnp.float32),
            pltpu.SemaphoreType.DMA,
        ],
    )
    def k(table_hbm, idx_hbm, out_hbm, idx_v, rows_v, sem):
        wid = lax.axis_index("s") * NC + lax.axis_index("c")
        base = wid * n_per_w
        for c in range(n_chunks):
            off = pl.multiple_of(base + c * chunk, 8)
            pltpu.sync_copy(idx_hbm.at[pl.ds(off, chunk)], idx_v)
            pltpu.async_copy(table_hbm.at[idx_v], rows_v, sem).wait()
            pltpu.sync_copy(rows_v, out_hbm.at[pl.ds(off, chunk)])

    return k(table, idx)


def _mlp1_body(hvi, he, hvj, w1a, w1b, w1c, w2t, w3t, b1, b2, b3, out):
    dh = jnp.dot(hvi[...], w1a[...], preferred_element_type=jnp.float32)
    dh += jnp.dot(he[...], w1b[...], preferred_element_type=jnp.float32)
    dh += jnp.dot(hvj[...], w1c[...], preferred_element_type=jnp.float32)
    dh += b1[...]
    dh = jnp.dot(dh, w2t[...], preferred_element_type=jnp.float32) + b2[...]
    dh = jnp.maximum(dh, 0.0)
    out[...] = jnp.dot(dh, w3t[...], preferred_element_type=jnp.float32) + b3[...]


def _run_mlp1(hvi, he, hvj, W1, b1, W2, b2, W3, b3, *, block=2000):
    W1t = W1.T
    w1a, w1b, w1c = W1t[0:D], W1t[D:2 * D], W1t[2 * D:3 * D]
    grid = (N_EDGES // block,)
    row_spec = pl.BlockSpec((block, D), lambda i: (i, 0))
    full = lambda s: pl.BlockSpec(s, lambda i: (0,) * len(s))
    return pl.pallas_call(
        _mlp1_body,
        grid=grid,
        in_specs=[row_spec, row_spec, row_spec,
                  full((D, D)), full((D, D)), full((D, D)),
                  full((D, D)), full((D, D)),
                  full((1, D)), full((1, D)), full((1, D))],
        out_specs=row_spec,
        out_shape=jax.ShapeDtypeStruct((N_EDGES, D), jnp.float32),
    )(hvi, he, hvj, w1a, w1b, w1c, W2.T, W3.T,
      b1.reshape(1, D), b2.reshape(1, D), b3.reshape(1, D))


def _normalize(x, gain, bias):
    mu = jnp.mean(x, axis=-1, keepdims=True)
    xc = x - mu
    var = jnp.sum(xc * xc, axis=-1, keepdims=True) * (1.0 / (D - 1))
    sigma = jnp.sqrt(var + EPS)
    return gain * xc / (sigma + EPS) + bias


def _tail_body(dh, dhrev, mask, he, wf1t, wf2t, bf1, bf2, g0, bn0, g1, bn1, out):
    m = mask[...]
    d = dh[...]
    merged = d + m * (0.5 * (dhrev[...] + d) - d)
    x = _normalize(he[...] + merged, g0[...], bn0[...])
    y = jnp.dot(x, wf1t[...], preferred_element_type=jnp.float32) + bf1[...]
    z = jnp.dot(y, wf2t[...], preferred_element_type=jnp.float32) + bf2[...]
    out[...] = _normalize(x + z, g1[...], bn1[...])


def _run_tail(dh, dhrev, mask, he, Wf1, bf1, Wf2, bf2, g0, bn0, g1, bn1, *, block=2000):
    grid = (N_EDGES // block,)
    row_spec = pl.BlockSpec((block, D), lambda i: (i, 0))
    mask_spec = pl.BlockSpec((block, 1), lambda i: (i, 0))
    full = lambda s: pl.BlockSpec(s, lambda i: (0,) * len(s))
    return pl.pallas_call(
        _tail_body,
        grid=grid,
        in_specs=[row_spec, row_spec, mask_spec, row_spec,
                  full((D, 2 * D)), full((2 * D, D)),
                  full((1, 2 * D)), full((1, D)),
                  full((1, D)), full((1, D)), full((1, D)), full((1, D))],
        out_specs=row_spec,
        out_shape=jax.ShapeDtypeStruct((N_EDGES, D), jnp.float32),
    )(dh, dhrev, mask, he, Wf1.T, Wf2.T,
      bf1.reshape(1, 2 * D), bf2.reshape(1, D),
      g0.reshape(1, D), bn0.reshape(1, D), g1.reshape(1, D), bn1.reshape(1, D))


def _partner_indices(src, dst):
    fwd = src * N_NODES + dst
    rev = dst * N_NODES + src
    order = jnp.argsort(fwd)
    sorted_fwd = fwd[order]
    pos = jnp.clip(jnp.searchsorted(sorted_fwd, rev), 0, N_EDGES - 1)
    has_rev = sorted_fwd[pos] == rev
    return order[pos].astype(jnp.int32), has_rev


def kernel(h_V, edge_index, h_E, W1, b1, W2, b2, W3, b3,
           Wf1, bf1, Wf2, bf2, g0, bn0, g1, bn1):
    src = edge_index[0]
    dst = edge_index[1]
    hvi = _sc_gather_rows(h_V, src)
    hvj = _sc_gather_rows(h_V, dst)
    partner, has_rev = _partner_indices(src, dst)
    dh = _run_mlp1(hvi, h_E, hvj, W1, b1, W2, b2, W3, b3)
    dhrev = _sc_gather_rows(dh, partner)
    mask = has_rev.astype(jnp.float32).reshape(N_EDGES, 1)
    return _run_tail(dh, dhrev, mask, h_E, Wf1, bf1, Wf2, bf2, g0, bn0, g1, bn1)
